# baseline (speedup 1.0000x reference)
import functools

import jax
import jax.numpy as jnp
from jax import lax
from jax.experimental import pallas as pl
from jax.experimental.pallas import tpu as pltpu
from jax.experimental.pallas import tpu_sc as plsc

_N = 10000
_E = 320000
_IN = 128
_H = 146
_HP = 160
_K = 7000
_NCONV = 4

_NC = 2
_NS = 16
_FS = _HP // _NC
_EPS = _E // _NS
_CH = 128
_NCHUNK = -(-_EPS // _CH)
_EPS_PAD = _NCHUNK * _CH
_NPAD = 10240


def _emb_l0_body(x_ref, wemb_ref, bemb_ref, w0_ref, out_ref):
    h0 = jnp.dot(x_ref[...], wemb_ref[...],
                 preferred_element_type=jnp.float32) + bemb_ref[...]
    hw = jnp.dot(h0, w0_ref[...], preferred_element_type=jnp.float32)
    out_ref[0] = hw[:, :_FS]
    out_ref[1] = hw[:, _FS:]


def _layer_body(p_ref, b_ref, w_ref, out_ref):
    agg = jnp.concatenate([p_ref[0], p_ref[1]], axis=1)
    h = jnp.maximum(agg + b_ref[...], 0.0)
    hw = jnp.dot(h, w_ref[...], preferred_element_type=jnp.float32)
    out_ref[0] = hw[:, :_FS]
    out_ref[1] = hw[:, _FS:]


def _topk_body(p_ref, b_ref, w_ref, out_ref):
    agg = jnp.concatenate([p_ref[0], p_ref[1]], axis=1)
    h = jnp.maximum(agg + b_ref[...], 0.0)
    w = w_ref[...]
    norm = jnp.sqrt(jnp.sum(w * w))
    sraw = lax.dot_general(w, h, (((1,), (1,)), ((), ())),
                           preferred_element_type=jnp.float32)
    score = jnp.tanh(sraw / norm)
    bits = lax.bitcast_convert_type(score, jnp.int32)
    key = jnp.where(bits >= 0, bits, bits ^ jnp.int32(0x7FFFFFFF))
    kk = jnp.int32(_K)
    int_min = jnp.int32(-(2 ** 31))
    idx = lax.broadcasted_iota(jnp.int32, (1, _NPAD), 1)
    key = jnp.where(idx < _N, key, int_min)
    cnt0 = jnp.sum((key >= 0).astype(jnp.int32))
    t0 = jnp.where(cnt0 >= kk, jnp.int32(0), int_min)

    def sstep(i, t):
        cand = t + (jnp.int32(1) << (jnp.int32(30) - i))
        cnt = jnp.sum((key >= cand).astype(jnp.int32))
        return jnp.where(cnt >= kk, cand, t)

    t = lax.fori_loop(0, 31, sstep, t0)
    c_gt = jnp.sum((key > t).astype(jnp.int32))
    r = kk - c_gt
    tie = key == t

    def istep(i, m):
        cand = m + (jnp.int32(1) << (jnp.int32(13) - i))
        f = jnp.sum((tie & (idx < cand)).astype(jnp.int32))
        return jnp.where(f <= r, cand, m)

    m = lax.fori_loop(0, 14, istep, jnp.int32(0))
    sel = (key > t) | (tie & (idx < m))
    wsel = jnp.where(sel, score, 0.0)
    out = jnp.dot(wsel, h, preferred_element_type=jnp.float32)
    out_ref[...] = out * (1.0 / _K)


def _tc_emb_l0(x, wemb, bemb, w0):
    return pl.pallas_call(
        _emb_l0_body,
        out_shape=jax.ShapeDtypeStruct((_NC, _NPAD, _FS), jnp.float32),
    )(x, wemb, bemb, w0)


def _tc_layer(p, b, w):
    return pl.pallas_call(
        _layer_body,
        out_shape=jax.ShapeDtypeStruct((_NC, _NPAD, _FS), jnp.float32),
    )(p, b, w)


def _tc_topk(p, b, w):
    return pl.pallas_call(
        _topk_body,
        out_shape=jax.ShapeDtypeStruct((1, _HP), jnp.float32),
    )(p, b, w)


def _sc_body(hw_hbm, src_hbm, dst_hbm, out_hbm, sidx_v, didx_v, rows_v,
             rows2_v, agg_sh, sem, sem2):
    c = lax.axis_index("c")
    s = lax.axis_index("s")
    tab = hw_hbm.at[c]
    pltpu.sync_copy(src_hbm.at[s], sidx_v)
    pltpu.sync_copy(dst_hbm.at[s], didx_v)

    def _zb(i, carry):
        rows_v[i // (_FS // 16), pl.ds((i % (_FS // 16)) * 16, 16)] = (
            jnp.zeros((16,), jnp.float32))
        return carry

    lax.fori_loop(0, _CH * (_FS // 16), _zb, 0)
    rows_per_tile = _NPAD // _NS
    for t in range(rows_per_tile // _CH):
        pltpu.sync_copy(
            rows_v, agg_sh.at[pl.ds(s * rows_per_tile + t * _CH, _CH)])
    plsc.subcore_barrier()

    pltpu.async_copy(tab.at[sidx_v.at[0]], rows_v, sem)

    def _pair(jj, carry):
        j = jj * 2
        pltpu.async_copy(tab.at[sidx_v.at[j + 1]], rows2_v, sem2)
        pltpu.make_async_copy(tab.at[sidx_v.at[j]], rows_v, sem).wait()
        pltpu.sync_copy(rows_v, agg_sh.at[didx_v.at[j]], add=True)
        pltpu.async_copy(tab.at[sidx_v.at[j + 2]], rows_v, sem)
        pltpu.make_async_copy(
            tab.at[sidx_v.at[j + 1]], rows2_v, sem2).wait()
        pltpu.sync_copy(rows2_v, agg_sh.at[didx_v.at[j + 1]], add=True)
        return carry

    lax.fori_loop(0, (_NCHUNK - 1) // 2, _pair, 0)
    pltpu.make_async_copy(
        tab.at[sidx_v.at[_NCHUNK - 1]], rows_v, sem).wait()
    pltpu.sync_copy(rows_v, agg_sh.at[didx_v.at[_NCHUNK - 1]], add=True)
    plsc.subcore_barrier()

    out_rows = _NPAD // _NS
    pltpu.sync_copy(agg_sh.at[pl.ds(s * out_rows, out_rows)],
                    out_hbm.at[c, pl.ds(s * out_rows, out_rows)])


@jax.jit
def _sc_scatter(hw, src_p, dst_p):
    mesh = plsc.VectorSubcoreMesh(core_axis_name="c", subcore_axis_name="s")
    k = functools.partial(
        pl.kernel,
        out_type=jax.ShapeDtypeStruct((_NC, _NPAD, _FS), jnp.float32),
        mesh=mesh,
        scratch_types=[
            pltpu.VMEM((_NCHUNK, _CH), jnp.int32),
            pltpu.VMEM((_NCHUNK, _CH), jnp.int32),
            pltpu.VMEM((_CH, _FS), jnp.float32),
            pltpu.VMEM((_CH, _FS), jnp.float32),
            pltpu.VMEM_SHARED((_NPAD, _FS), jnp.float32),
            pltpu.SemaphoreType.DMA,
            pltpu.SemaphoreType.DMA,
        ],
        compiler_params=pltpu.CompilerParams(use_tc_tiling_on_sc=False),
    )(_sc_body)
    return k(hw, src_p, dst_p)


def _prep_edges(edge_index):
    src = edge_index[0].reshape(_NS, _EPS)
    dst = edge_index[1].reshape(_NS, _EPS)
    pad = _EPS_PAD - _EPS
    pad_src = jnp.zeros((_NS, pad), jnp.int32)
    off = (jnp.arange(_NS, dtype=jnp.int32)[:, None] * pad
           + jnp.arange(pad, dtype=jnp.int32)[None, :])
    pad_dst = _N + off % (_NPAD - _N)
    src_p = jnp.concatenate([src, pad_src], axis=1).reshape(_NS, _NCHUNK, _CH)
    dst_p = jnp.concatenate([dst, pad_dst], axis=1).reshape(_NS, _NCHUNK, _CH)
    return src_p, dst_p


def kernel(x, edge_index, batch, W_emb, b_emb, conv_W, conv_b, topk_w):
    f32 = jnp.float32
    wemb = jnp.pad(W_emb.astype(f32), ((0, 0), (0, _HP - _H)))
    bemb = jnp.pad(b_emb.astype(f32), (0, _HP - _H)).reshape(1, _HP)
    convw = jnp.pad(conv_W.astype(f32), ((0, 0), (0, _HP - _H), (0, _HP - _H)))
    convb = jnp.pad(conv_b.astype(f32), ((0, 0), (0, _HP - _H)))
    wtop = jnp.pad(topk_w.astype(f32), (0, _HP - _H)).reshape(1, _HP)

    src_p, dst_p = _prep_edges(edge_index)

    x_p = jnp.pad(x.astype(f32), ((0, _NPAD - _N), (0, 0)))
    hw = _tc_emb_l0(x_p, wemb, bemb, convw[0])
    out = None
    for li in range(_NCONV):
        p = _sc_scatter(hw, src_p, dst_p)
        b = convb[li].reshape(1, _HP)
        if li + 1 < _NCONV:
            hw = _tc_layer(p, b, convw[li + 1])
        else:
            out = _tc_topk(p, b, wtop)
    return out[:, :_H]


# speedup vs baseline: 1.0046x; 1.0046x over previous
import functools

import jax
import jax.numpy as jnp
from jax import lax
from jax.experimental import pallas as pl
from jax.experimental.pallas import tpu as pltpu
from jax.experimental.pallas import tpu_sc as plsc

_N = 10000
_E = 320000
_IN = 128
_H = 146
_HP = 160
_K = 7000
_NCONV = 4

_NC = 2
_NS = 16
_FS = _HP // _NC
_EPS = _E // _NS
_CH = 128
_NCHUNK = -(-_EPS // _CH)
_EPS_PAD = _NCHUNK * _CH
_NPAD = 10240


def _emb_l0_body(x_ref, wemb_ref, bemb_ref, w0_ref, out_ref):
    h0 = jnp.dot(x_ref[...], wemb_ref[...],
                 preferred_element_type=jnp.float32) + bemb_ref[...]
    hw = jnp.dot(h0, w0_ref[...], preferred_element_type=jnp.float32)
    z = jnp.zeros((_NPAD - _N, _FS), jnp.float32)
    out_ref[0] = jnp.concatenate([hw[:, :_FS], z], axis=0)
    out_ref[1] = jnp.concatenate([hw[:, _FS:], z], axis=0)


def _layer_body(p_ref, b_ref, w_ref, out_ref):
    agg = jnp.concatenate([p_ref[0], p_ref[1]], axis=1)
    h = jnp.maximum(agg + b_ref[...], 0.0)
    hw = jnp.dot(h, w_ref[...], preferred_element_type=jnp.float32)
    out_ref[0] = hw[:, :_FS]
    out_ref[1] = hw[:, _FS:]


def _topk_body(p_ref, b_ref, w_ref, out_ref):
    agg = jnp.concatenate([p_ref[0], p_ref[1]], axis=1)
    h = jnp.maximum(agg + b_ref[...], 0.0)
    w = w_ref[...]
    norm = jnp.sqrt(jnp.sum(w * w))
    sraw = lax.dot_general(w, h, (((1,), (1,)), ((), ())),
                           preferred_element_type=jnp.float32)
    score = jnp.tanh(sraw / norm)
    bits = lax.bitcast_convert_type(score, jnp.int32)
    key = jnp.where(bits >= 0, bits, bits ^ jnp.int32(0x7FFFFFFF))
    kk = jnp.int32(_K)
    int_min = jnp.int32(-(2 ** 31))
    idx = lax.broadcasted_iota(jnp.int32, (1, _NPAD), 1)
    key = jnp.where(idx < _N, key, int_min)
    cnt0 = jnp.sum((key >= 0).astype(jnp.int32))
    t0 = jnp.where(cnt0 >= kk, jnp.int32(0), int_min)

    def sstep(i, t):
        cand = t + (jnp.int32(1) << (jnp.int32(30) - i))
        cnt = jnp.sum((key >= cand).astype(jnp.int32))
        return jnp.where(cnt >= kk, cand, t)

    t = lax.fori_loop(0, 31, sstep, t0)
    c_gt = jnp.sum((key > t).astype(jnp.int32))
    r = kk - c_gt
    tie = key == t

    def istep(i, m):
        cand = m + (jnp.int32(1) << (jnp.int32(13) - i))
        f = jnp.sum((tie & (idx < cand)).astype(jnp.int32))
        return jnp.where(f <= r, cand, m)

    m = lax.fori_loop(0, 14, istep, jnp.int32(0))
    sel = (key > t) | (tie & (idx < m))
    wsel = jnp.where(sel, score, 0.0)
    out = jnp.dot(wsel, h, preferred_element_type=jnp.float32)
    out_ref[...] = out * (1.0 / _K)


def _tc_emb_l0(x, wemb, bemb, w0):
    return pl.pallas_call(
        _emb_l0_body,
        out_shape=jax.ShapeDtypeStruct((_NC, _NPAD, _FS), jnp.float32),
    )(x, wemb, bemb, w0)


def _tc_layer(p, b, w):
    return pl.pallas_call(
        _layer_body,
        out_shape=jax.ShapeDtypeStruct((_NC, _NPAD, _FS), jnp.float32),
    )(p, b, w)


def _tc_topk(p, b, w):
    return pl.pallas_call(
        _topk_body,
        out_shape=jax.ShapeDtypeStruct((1, _HP), jnp.float32),
    )(p, b, w)


def _sc_body(hw_hbm, src_hbm, dst_hbm, out_hbm, sidx_v, didx_v, rows_v,
             rows2_v, agg_sh, sem, sem2):
    c = lax.axis_index("c")
    s = lax.axis_index("s")
    tab = hw_hbm.at[c]
    pltpu.sync_copy(src_hbm.at[s], sidx_v)
    pltpu.sync_copy(dst_hbm.at[s], didx_v)

    def _zb(i, carry):
        rows_v[i // (_FS // 16), pl.ds((i % (_FS // 16)) * 16, 16)] = (
            jnp.zeros((16,), jnp.float32))
        return carry

    lax.fori_loop(0, _CH * (_FS // 16), _zb, 0)
    rows_per_tile = _NPAD // _NS
    for t in range(rows_per_tile // _CH):
        pltpu.sync_copy(
            rows_v, agg_sh.at[pl.ds(s * rows_per_tile + t * _CH, _CH)])
    plsc.subcore_barrier()

    pltpu.async_copy(tab.at[sidx_v.at[0]], rows_v, sem)

    def _pair(jj, carry):
        j = jj * 2
        pltpu.async_copy(tab.at[sidx_v.at[j + 1]], rows2_v, sem2)
        pltpu.make_async_copy(tab.at[sidx_v.at[j]], rows_v, sem).wait()
        pltpu.sync_copy(rows_v, agg_sh.at[didx_v.at[j]], add=True)
        pltpu.async_copy(tab.at[sidx_v.at[j + 2]], rows_v, sem)
        pltpu.make_async_copy(
            tab.at[sidx_v.at[j + 1]], rows2_v, sem2).wait()
        pltpu.sync_copy(rows2_v, agg_sh.at[didx_v.at[j + 1]], add=True)
        return carry

    lax.fori_loop(0, (_NCHUNK - 1) // 2, _pair, 0)
    pltpu.make_async_copy(
        tab.at[sidx_v.at[_NCHUNK - 1]], rows_v, sem).wait()
    pltpu.sync_copy(rows_v, agg_sh.at[didx_v.at[_NCHUNK - 1]], add=True)
    plsc.subcore_barrier()

    out_rows = _NPAD // _NS
    pltpu.sync_copy(agg_sh.at[pl.ds(s * out_rows, out_rows)],
                    out_hbm.at[c, pl.ds(s * out_rows, out_rows)])


@jax.jit
def _sc_scatter(hw, src_p, dst_p):
    mesh = plsc.VectorSubcoreMesh(core_axis_name="c", subcore_axis_name="s")
    k = functools.partial(
        pl.kernel,
        out_type=jax.ShapeDtypeStruct((_NC, _NPAD, _FS), jnp.float32),
        mesh=mesh,
        scratch_types=[
            pltpu.VMEM((_NCHUNK, _CH), jnp.int32),
            pltpu.VMEM((_NCHUNK, _CH), jnp.int32),
            pltpu.VMEM((_CH, _FS), jnp.float32),
            pltpu.VMEM((_CH, _FS), jnp.float32),
            pltpu.VMEM_SHARED((_NPAD, _FS), jnp.float32),
            pltpu.SemaphoreType.DMA,
            pltpu.SemaphoreType.DMA,
        ],
        compiler_params=pltpu.CompilerParams(use_tc_tiling_on_sc=False),
    )(_sc_body)
    return k(hw, src_p, dst_p)


def _prep_edges(edge_index):
    src = edge_index[0].reshape(_NS, _EPS)
    dst = edge_index[1].reshape(_NS, _EPS)
    pad = _EPS_PAD - _EPS
    pad_src = jnp.zeros((_NS, pad), jnp.int32)
    off = (jnp.arange(_NS, dtype=jnp.int32)[:, None] * pad
           + jnp.arange(pad, dtype=jnp.int32)[None, :])
    pad_dst = _N + off % (_NPAD - _N)
    src_p = jnp.concatenate([src, pad_src], axis=1).reshape(_NS, _NCHUNK, _CH)
    dst_p = jnp.concatenate([dst, pad_dst], axis=1).reshape(_NS, _NCHUNK, _CH)
    return src_p, dst_p


def kernel(x, edge_index, batch, W_emb, b_emb, conv_W, conv_b, topk_w):
    f32 = jnp.float32
    wemb = jnp.pad(W_emb.astype(f32), ((0, 0), (0, _HP - _H)))
    bemb = jnp.pad(b_emb.astype(f32), (0, _HP - _H)).reshape(1, _HP)
    convw = jnp.pad(conv_W.astype(f32), ((0, 0), (0, _HP - _H), (0, _HP - _H)))
    convb = jnp.pad(conv_b.astype(f32), ((0, 0), (0, _HP - _H)))
    wtop = jnp.pad(topk_w.astype(f32), (0, _HP - _H)).reshape(1, _HP)

    src_p, dst_p = _prep_edges(edge_index)

    hw = _tc_emb_l0(x.astype(f32), wemb, bemb, convw[0])
    out = None
    for li in range(_NCONV):
        p = _sc_scatter(hw, src_p, dst_p)
        b = convb[li].reshape(1, _HP)
        if li + 1 < _NCONV:
            hw = _tc_layer(p, b, convw[li + 1])
        else:
            out = _tc_topk(p, b, wtop)
    return out[:, :_H]


# speedup vs baseline: 1.0248x; 1.0201x over previous
import functools

import jax
import jax.numpy as jnp
from jax import lax
from jax.experimental import pallas as pl
from jax.experimental.pallas import tpu as pltpu
from jax.experimental.pallas import tpu_sc as plsc

_N = 10000
_E = 320000
_IN = 128
_H = 146
_HP = 160
_K = 7000
_NCONV = 4

_NC = 2
_NS = 16
_FS = _HP // _NC
_EPS = _E // _NS
_CH = 128
_NCHUNK = -(-_EPS // _CH)
_EPS_PAD = _NCHUNK * _CH
_NPAD = 10240


def _emb_l0_body(x_ref, wemb_ref, bemb_ref, w0_ref, out_ref):
    h0 = jnp.dot(x_ref[...], wemb_ref[...],
                 preferred_element_type=jnp.float32) + bemb_ref[...]
    hw = jnp.dot(h0, w0_ref[...], preferred_element_type=jnp.float32)
    z = jnp.zeros((_NPAD - _N, _FS), jnp.float32)
    out_ref[0] = jnp.concatenate([hw[:, :_FS], z], axis=0)
    out_ref[1] = jnp.concatenate([hw[:, _FS:], z], axis=0)


def _layer_body(p_ref, b_ref, w_ref, out_ref):
    agg = jnp.concatenate([p_ref[0], p_ref[1]], axis=1)
    h = jnp.maximum(agg + b_ref[...], 0.0)
    hw = jnp.dot(h, w_ref[...], preferred_element_type=jnp.float32)
    out_ref[0] = hw[:, :_FS]
    out_ref[1] = hw[:, _FS:]


def _topk_body(p_ref, b_ref, w_ref, out_ref):
    agg = jnp.concatenate([p_ref[0], p_ref[1]], axis=1)
    h = jnp.maximum(agg + b_ref[...], 0.0)
    w = w_ref[...]
    norm = jnp.sqrt(jnp.sum(w * w))
    sraw = lax.dot_general(w, h, (((1,), (1,)), ((), ())),
                           preferred_element_type=jnp.float32)
    score = jnp.tanh(sraw / norm)
    bits = lax.bitcast_convert_type(score, jnp.int32)
    key = jnp.where(bits >= 0, bits, bits ^ jnp.int32(0x7FFFFFFF))
    kk = jnp.int32(_K)
    int_min = jnp.int32(-(2 ** 31))
    idx = lax.broadcasted_iota(jnp.int32, (1, _NPAD), 1)
    key = jnp.where(idx < _N, key, int_min)
    cnt0 = jnp.sum((key >= 0).astype(jnp.int32))
    t0 = jnp.where(cnt0 >= kk, jnp.int32(0), int_min)

    def sstep(i, t):
        cand = t + (jnp.int32(1) << (jnp.int32(30) - i))
        cnt = jnp.sum((key >= cand).astype(jnp.int32))
        return jnp.where(cnt >= kk, cand, t)

    t = lax.fori_loop(0, 31, sstep, t0)
    c_gt = jnp.sum((key > t).astype(jnp.int32))
    r = kk - c_gt
    tie = key == t

    def istep(i, m):
        cand = m + (jnp.int32(1) << (jnp.int32(13) - i))
        f = jnp.sum((tie & (idx < cand)).astype(jnp.int32))
        return jnp.where(f <= r, cand, m)

    m = lax.fori_loop(0, 14, istep, jnp.int32(0))
    sel = (key > t) | (tie & (idx < m))
    wsel = jnp.where(sel, score, 0.0)
    out = jnp.dot(wsel, h, preferred_element_type=jnp.float32)
    out_ref[...] = out * (1.0 / _K)


def _tc_emb_l0(x, wemb, bemb, w0):
    return pl.pallas_call(
        _emb_l0_body,
        out_shape=jax.ShapeDtypeStruct((_NC, _NPAD, _FS), jnp.float32),
    )(x, wemb, bemb, w0)


def _tc_layer(p, b, w):
    return pl.pallas_call(
        _layer_body,
        out_shape=jax.ShapeDtypeStruct((_NC, _NPAD, _FS), jnp.float32),
    )(p, b, w)


def _tc_topk(p, b, w):
    return pl.pallas_call(
        _topk_body,
        out_shape=jax.ShapeDtypeStruct((1, _HP), jnp.float32),
    )(p, b, w)


def _sc_body(hw_hbm, src_hbm, dst_hbm, out_hbm, sidx_v, didx_v, rows_v,
             rows2_v, agg_sh, sem, sem2):
    c = lax.axis_index("c")
    s = lax.axis_index("s")
    tab = hw_hbm.at[c]
    pltpu.async_copy(src_hbm.at[s], sidx_v, sem)
    pltpu.async_copy(dst_hbm.at[s], didx_v, sem2)

    def _zb(i, carry):
        rows2_v[i // (_FS // 16), pl.ds((i % (_FS // 16)) * 16, 16)] = (
            jnp.zeros((16,), jnp.float32))
        return carry

    lax.fori_loop(0, _CH * (_FS // 16), _zb, 0)
    pltpu.make_async_copy(src_hbm.at[s], sidx_v, sem).wait()
    pltpu.async_copy(tab.at[sidx_v.at[0]], rows_v, sem)
    rows_per_tile = _NPAD // _NS
    for t in range(rows_per_tile // _CH):
        pltpu.sync_copy(
            rows2_v, agg_sh.at[pl.ds(s * rows_per_tile + t * _CH, _CH)])
    pltpu.make_async_copy(dst_hbm.at[s], didx_v, sem2).wait()
    plsc.subcore_barrier()


    def _pair(jj, carry):
        j = jj * 2
        pltpu.async_copy(tab.at[sidx_v.at[j + 1]], rows2_v, sem2)
        pltpu.make_async_copy(tab.at[sidx_v.at[j]], rows_v, sem).wait()
        pltpu.sync_copy(rows_v, agg_sh.at[didx_v.at[j]], add=True)
        pltpu.async_copy(tab.at[sidx_v.at[j + 2]], rows_v, sem)
        pltpu.make_async_copy(
            tab.at[sidx_v.at[j + 1]], rows2_v, sem2).wait()
        pltpu.sync_copy(rows2_v, agg_sh.at[didx_v.at[j + 1]], add=True)
        return carry

    lax.fori_loop(0, (_NCHUNK - 1) // 2, _pair, 0)
    pltpu.make_async_copy(
        tab.at[sidx_v.at[_NCHUNK - 1]], rows_v, sem).wait()
    pltpu.sync_copy(rows_v, agg_sh.at[didx_v.at[_NCHUNK - 1]], add=True)
    plsc.subcore_barrier()

    out_rows = _NPAD // _NS
    pltpu.sync_copy(agg_sh.at[pl.ds(s * out_rows, out_rows)],
                    out_hbm.at[c, pl.ds(s * out_rows, out_rows)])


@jax.jit
def _sc_scatter(hw, src_p, dst_p):
    mesh = plsc.VectorSubcoreMesh(core_axis_name="c", subcore_axis_name="s")
    k = functools.partial(
        pl.kernel,
        out_type=jax.ShapeDtypeStruct((_NC, _NPAD, _FS), jnp.float32),
        mesh=mesh,
        scratch_types=[
            pltpu.VMEM((_NCHUNK, _CH), jnp.int32),
            pltpu.VMEM((_NCHUNK, _CH), jnp.int32),
            pltpu.VMEM((_CH, _FS), jnp.float32),
            pltpu.VMEM((_CH, _FS), jnp.float32),
            pltpu.VMEM_SHARED((_NPAD, _FS), jnp.float32),
            pltpu.SemaphoreType.DMA,
            pltpu.SemaphoreType.DMA,
        ],
        compiler_params=pltpu.CompilerParams(use_tc_tiling_on_sc=False),
    )(_sc_body)
    return k(hw, src_p, dst_p)


def _prep_edges(edge_index):
    src = edge_index[0].reshape(_NS, _EPS)
    dst = edge_index[1].reshape(_NS, _EPS)
    pad = _EPS_PAD - _EPS
    pad_src = jnp.zeros((_NS, pad), jnp.int32)
    off = (jnp.arange(_NS, dtype=jnp.int32)[:, None] * pad
           + jnp.arange(pad, dtype=jnp.int32)[None, :])
    pad_dst = _N + off % (_NPAD - _N)
    src_p = jnp.concatenate([src, pad_src], axis=1).reshape(_NS, _NCHUNK, _CH)
    dst_p = jnp.concatenate([dst, pad_dst], axis=1).reshape(_NS, _NCHUNK, _CH)
    return src_p, dst_p


def kernel(x, edge_index, batch, W_emb, b_emb, conv_W, conv_b, topk_w):
    f32 = jnp.float32
    wemb = jnp.pad(W_emb.astype(f32), ((0, 0), (0, _HP - _H)))
    bemb = jnp.pad(b_emb.astype(f32), (0, _HP - _H)).reshape(1, _HP)
    convw = jnp.pad(conv_W.astype(f32), ((0, 0), (0, _HP - _H), (0, _HP - _H)))
    convb = jnp.pad(conv_b.astype(f32), ((0, 0), (0, _HP - _H)))
    wtop = jnp.pad(topk_w.astype(f32), (0, _HP - _H)).reshape(1, _HP)

    src_p, dst_p = _prep_edges(edge_index)

    hw = _tc_emb_l0(x.astype(f32), wemb, bemb, convw[0])
    out = None
    for li in range(_NCONV):
        p = _sc_scatter(hw, src_p, dst_p)
        b = convb[li].reshape(1, _HP)
        if li + 1 < _NCONV:
            hw = _tc_layer(p, b, convw[li + 1])
        else:
            out = _tc_topk(p, b, wtop)
    return out[:, :_H]
